# speedup vs baseline: 2.9728x; 2.9728x over previous
import functools

import jax
import jax.numpy as jnp
from jax import lax
from jax.experimental import pallas as pl
from jax.experimental.pallas import tpu as pltpu
from jax.experimental.pallas import tpu_sc as plsc

NC = 2
NS = 16
NW = NC * NS
CHUNK = 128


@functools.lru_cache(maxsize=None)
def _make_gather(vocab, d, total):
    assert total % (NW * CHUNK) == 0
    b_per_w = total // NW
    n_chunks = b_per_w // CHUNK
    mesh = plsc.VectorSubcoreMesh(core_axis_name="c", subcore_axis_name="s")

    @functools.partial(
        pl.kernel,
        mesh=mesh,
        out_type=jax.ShapeDtypeStruct((total, d), jnp.float32),
        scratch_types=[
            pltpu.VMEM((n_chunks, CHUNK), jnp.int32),
            pltpu.VMEM((CHUNK, d), jnp.float32),
            pltpu.SemaphoreType.DMA,
        ],
    )
    def gather_kernel(table_hbm, idx_hbm, out_hbm, idx_v, rows_v, sem):
        wid = lax.axis_index("s") * NC + lax.axis_index("c")
        base = wid * b_per_w
        pltpu.sync_copy(idx_hbm.at[wid], idx_v)

        def body(g, carry):
            pltpu.async_copy(table_hbm.at[idx_v.at[g]], rows_v, sem).wait()
            pltpu.sync_copy(rows_v, out_hbm.at[pl.ds(base + g * CHUNK, CHUNK)])
            return carry

        lax.fori_loop(0, n_chunks, body, 0)

    return gather_kernel


def kernel(weights, token_ids):
    b, s = token_ids.shape
    vocab, d = weights.shape
    total = b * s
    ids = token_ids.reshape(-1).astype(jnp.int32)
    ids3 = ids.reshape(NW, total // NW // CHUNK, CHUNK)
    out = _make_gather(vocab, d, total)(weights, ids3)
    return out.reshape(b, s, d)


# speedup vs baseline: 3.3434x; 1.1247x over previous
import functools

import jax
import jax.numpy as jnp
from jax import lax
from jax.experimental import pallas as pl
from jax.experimental.pallas import tpu as pltpu
from jax.experimental.pallas import tpu_sc as plsc

NC = 2
NS = 16
NW = NC * NS
CHUNK = 128
NBUF = 5


@functools.lru_cache(maxsize=None)
def _make_gather(vocab, d, total):
    assert total % (NW * CHUNK) == 0
    b_per_w = total // NW
    n_chunks = b_per_w // CHUNK
    assert n_chunks % NBUF == 0 and n_chunks >= NBUF
    mesh = plsc.VectorSubcoreMesh(core_axis_name="c", subcore_axis_name="s")

    @functools.partial(
        pl.kernel,
        mesh=mesh,
        out_type=jax.ShapeDtypeStruct((total, d), jnp.float32),
        scratch_types=[
            pltpu.VMEM((n_chunks, CHUNK), jnp.int32),
            pltpu.VMEM((NBUF, CHUNK, d), jnp.float32),
        ]
        + [pltpu.SemaphoreType.DMA] * (2 * NBUF),
    )
    def gather_kernel(table_hbm, idx_hbm, out_hbm, idx_v, rows_v, *sems):
        gsem = sems[:NBUF]
        osem = sems[NBUF:]
        wid = lax.axis_index("s") * NC + lax.axis_index("c")
        base = wid * b_per_w
        pltpu.sync_copy(idx_hbm.at[wid], idx_v)

        def start_gather(g, b):
            pltpu.async_copy(table_hbm.at[idx_v.at[g]], rows_v.at[b], gsem[b])

        def wait_gather(g, b):
            pltpu.make_async_copy(
                table_hbm.at[idx_v.at[g]], rows_v.at[b], gsem[b]
            ).wait()

        def start_out(g, b):
            pltpu.async_copy(
                rows_v.at[b], out_hbm.at[pl.ds(base + g * CHUNK, CHUNK)], osem[b]
            )

        def wait_out(g, b):
            pltpu.make_async_copy(
                rows_v.at[b], out_hbm.at[pl.ds(base + g * CHUNK, CHUNK)], osem[b]
            ).wait()

        for c in range(NBUF - 1):
            start_gather(c, c)

        def outer(i, carry):
            go = i * NBUF
            for b in range(NBUF):
                g = go + b
                wait_gather(g, b)
                start_out(g, b)
                bn = (b + NBUF - 1) % NBUF
                gn = g + NBUF - 1

                @pl.when(g >= 1)
                def _():
                    wait_out(g - 1, bn)

                @pl.when(gn < n_chunks)
                def _():
                    start_gather(gn, bn)

            return carry

        lax.fori_loop(0, n_chunks // NBUF, outer, 0)
        wait_out(n_chunks - 1, (n_chunks - 1) % NBUF)

    return gather_kernel


def kernel(weights, token_ids):
    b, s = token_ids.shape
    vocab, d = weights.shape
    total = b * s
    ids = token_ids.reshape(-1).astype(jnp.int32)
    ids3 = ids.reshape(NW, total // NW // CHUNK, CHUNK)
    out = _make_gather(vocab, d, total)(weights, ids3)
    return out.reshape(b, s, d)


# speedup vs baseline: 6.0262x; 1.8024x over previous
import functools

import jax
import jax.numpy as jnp
from jax import lax
from jax.experimental import pallas as pl
from jax.experimental.pallas import tpu as pltpu
from jax.experimental.pallas import tpu_sc as plsc

NC = 2
NS = 16
NW = NC * NS
ROWS = 2
NBUF = 8


@functools.lru_cache(maxsize=None)
def _make_gather(vocab, d, batch, seq):
    assert batch % (NW * ROWS) == 0
    b_per_w = batch // NW
    n_chunks = b_per_w // ROWS
    cids = ROWS * seq
    assert cids <= 128
    assert n_chunks % NBUF == 0 and n_chunks >= NBUF
    mesh = plsc.VectorSubcoreMesh(core_axis_name="c", subcore_axis_name="s")

    @functools.partial(
        pl.kernel,
        mesh=mesh,
        out_type=jax.ShapeDtypeStruct((batch, seq, d), jnp.float32),
        scratch_types=[
            pltpu.VMEM((n_chunks, cids), jnp.int32),
            pltpu.VMEM((NBUF, cids, d), jnp.float32),
        ]
        + [pltpu.SemaphoreType.DMA] * (2 * NBUF),
    )
    def gather_kernel(table_hbm, idx_hbm, out_hbm, idx_v, rows_v, *sems):
        gsem = sems[:NBUF]
        osem = sems[NBUF:]
        wid = lax.axis_index("s") * NC + lax.axis_index("c")
        base = wid * b_per_w
        pltpu.sync_copy(idx_hbm.at[wid], idx_v)

        def start_gather(g, b):
            pltpu.async_copy(table_hbm.at[idx_v.at[g]], rows_v.at[b], gsem[b])

        def wait_gather(g, b):
            pltpu.make_async_copy(
                table_hbm.at[idx_v.at[g]], rows_v.at[b], gsem[b]
            ).wait()

        def out_copies(g, b):
            for r in range(ROWS):
                yield (
                    rows_v.at[b, pl.ds(r * seq, seq)],
                    out_hbm.at[base + g * ROWS + r],
                    osem[b],
                )

        def start_out(g, b):
            for src, dst, sem in out_copies(g, b):
                pltpu.async_copy(src, dst, sem)

        def wait_out(g, b):
            for src, dst, sem in out_copies(g, b):
                pltpu.make_async_copy(src, dst, sem).wait()

        for c in range(NBUF - 1):
            start_gather(c, c)

        def outer(i, carry):
            go = i * NBUF
            for b in range(NBUF):
                g = go + b
                wait_gather(g, b)
                start_out(g, b)
                bn = (b + NBUF - 1) % NBUF
                gn = g + NBUF - 1

                @pl.when(g >= 1)
                def _():
                    wait_out(g - 1, bn)

                @pl.when(gn < n_chunks)
                def _():
                    start_gather(gn, bn)

            return carry

        lax.fori_loop(0, n_chunks // NBUF, outer, 0)
        wait_out(n_chunks - 1, (n_chunks - 1) % NBUF)

    return gather_kernel


def kernel(weights, token_ids):
    batch, seq = token_ids.shape
    vocab, d = weights.shape
    ids = token_ids.astype(jnp.int32)
    ids3 = ids.reshape(NW, batch // NW // ROWS, ROWS * seq)
    return _make_gather(vocab, d, batch, seq)(weights, ids3)
